# baseline (speedup 1.0000x reference)
import functools

import jax
import jax.numpy as jnp
from jax import lax
from jax.experimental import pallas as pl
from jax.experimental.pallas import tpu as pltpu
from jax.experimental.pallas import tpu_sc as plsc

_NC = 2
_NS = 16
_NW = _NC * _NS

_NUNIT = 200 * 8
_PER_W = _NUNIT // _NW
_NBUF = 4


def _make_kernel():
    mesh = plsc.VectorSubcoreMesh(
        core_axis_name="c", subcore_axis_name="s",
        num_cores=_NC, num_subcores=_NS,
    )

    scratch = (
        [pltpu.VMEM((128,), jnp.int32) for _ in range(_NBUF)]
        + [pltpu.VMEM((128, 32), jnp.float32) for _ in range(_NBUF)]
        + [pltpu.VMEM((4, 8, 128), jnp.float32) for _ in range(_NBUF)]
        + [pltpu.SemaphoreType.DMA for _ in range(3 * _NBUF)]
    )

    @functools.partial(
        pl.kernel,
        mesh=mesh,
        out_type=jax.ShapeDtypeStruct((400, 4, 8, 8, 128), jnp.float32),
        scratch_types=scratch,
        compiler_params=pltpu.CompilerParams(use_tc_tiling_on_sc=False,
                                             needs_layout_passes=False),
    )
    def gather_expand(table, idx4, out5, *refs):
        idxv = refs[0:_NBUF]
        buf = refs[_NBUF:2 * _NBUF]
        tr = refs[2 * _NBUF:3 * _NBUF]
        isem = refs[3 * _NBUF:4 * _NBUF]
        gsem = refs[4 * _NBUF:5 * _NBUF]
        wsem = refs[5 * _NBUF:6 * _NBUF]

        wid = lax.axis_index("s") * _NC + lax.axis_index("c")
        base = wid * _PER_W
        last = base + _PER_W - 1

        rows = [lax.iota(jnp.int32, 16) + (lg * 16) for lg in range(8)]
        cols = [jnp.full((16,), c, jnp.int32) for c in range(32)]

        def icopy_start(u, p):
            s, bt = u // 8, u % 8
            return pltpu.async_copy(idx4.at[s // 8, bt, s % 8, :], idxv[p],
                                    isem[p])

        def gstart(p):
            return pltpu.async_copy(table.at[idxv[p]], buf[p], gsem[p])

        def gwait(p):
            pltpu.make_async_copy(table.at[idxv[p]], buf[p], gsem[p]).wait()

        def transpose(p):
            for c in range(32):
                for lg in range(8):
                    vals = plsc.load_gather(buf[p], [rows[lg], cols[c]])
                    tr[p][c // 8, c % 8, pl.ds(lg * 16, 16)] = vals

        def wstart(u, p):
            s, bt = u // 8, u % 8
            pltpu.async_copy(tr[p], out5.at[2 * s, :, bt, :, :], wsem[p])
            pltpu.async_copy(tr[p], out5.at[2 * s + 1, :, bt, :, :], wsem[p])

        def wwait(u, p):
            s, bt = u // 8, u % 8
            for _ in range(2):
                pltpu.make_async_copy(
                    tr[p], out5.at[2 * s, :, bt, :, :], wsem[p]).wait()

        ics = [icopy_start(base + p, p) for p in range(_NBUF)]
        for p in range(_NBUF):
            ics[p].wait()
            gstart(p)

        def body(k, carry):
            for p in range(_NBUF):
                u = jnp.minimum(base + k * _NBUF + p, last)
                nxt = jnp.minimum(base + k * _NBUF + p + _NBUF, last)
                gwait(p)
                ic = icopy_start(nxt, p)

                @pl.when(k >= 1)
                def _():
                    wwait(u, p)

                transpose(p)
                wstart(u, p)
                ic.wait()
                gstart(p)
            return carry

        nk = (_PER_W + _NBUF - 1) // _NBUF
        lax.fori_loop(0, nk, body, 0)

        for p in range(_NBUF):
            gwait(p)
            wwait(last, p)

    return gather_expand


_GATHER_EXPAND = _make_kernel()


def kernel(input, embedding_weight):
    idx4 = jnp.transpose(input.astype(jnp.int32).reshape(8, 128, 25, 8),
                         (2, 0, 3, 1))
    out5 = _GATHER_EXPAND(embedding_weight, idx4)
    return out5.transpose(2, 4, 0, 1, 3).reshape(1024, 400, 32)


# speedup vs baseline: 1.0518x; 1.0518x over previous
import functools

import jax
import jax.numpy as jnp
from jax import lax
from jax.experimental import pallas as pl
from jax.experimental.pallas import tpu as pltpu
from jax.experimental.pallas import tpu_sc as plsc

_NC = 2
_NS = 16
_NW = _NC * _NS

_NUNIT = 200 * 8
_PER_W = _NUNIT // _NW


def _make_kernel():
    mesh = plsc.VectorSubcoreMesh(
        core_axis_name="c", subcore_axis_name="s",
        num_cores=_NC, num_subcores=_NS,
    )

    @functools.partial(
        pl.kernel,
        mesh=mesh,
        out_type=jax.ShapeDtypeStruct((400, 4, 8, 8, 128), jnp.float32),
        scratch_types=[
            pltpu.VMEM((128,), jnp.int32),
            pltpu.VMEM((128,), jnp.int32),
            pltpu.VMEM((128, 32), jnp.float32),
            pltpu.VMEM((128, 32), jnp.float32),
            pltpu.VMEM((4, 8, 128), jnp.float32),
            pltpu.VMEM((4, 8, 128), jnp.float32),
            pltpu.SemaphoreType.DMA,
            pltpu.SemaphoreType.DMA,
            pltpu.SemaphoreType.DMA,
            pltpu.SemaphoreType.DMA,
            pltpu.SemaphoreType.DMA,
            pltpu.SemaphoreType.DMA,
        ],
        compiler_params=pltpu.CompilerParams(use_tc_tiling_on_sc=False,
                                             needs_layout_passes=False),
    )
    def gather_expand(table, idx4, out5, idxv0, idxv1, buf0, buf1, tr0, tr1,
                      isem0, isem1, gsem0, gsem1, wsem0, wsem1):
        wid = lax.axis_index("s") * _NC + lax.axis_index("c")
        base = wid * _PER_W

        idxv = (idxv0, idxv1)
        buf = (buf0, buf1)
        tr = (tr0, tr1)
        isem = (isem0, isem1)
        gsem = (gsem0, gsem1)
        wsem = (wsem0, wsem1)

        rows = [lax.iota(jnp.int32, 16) + (lg * 16) for lg in range(8)]
        cols = [jnp.full((16,), c, jnp.int32) for c in range(32)]

        def icopy_start(u, p):
            s, bt = u // 8, u % 8
            return pltpu.async_copy(idx4.at[s // 8, bt, s % 8, :], idxv[p],
                                    isem[p])

        def gstart(p):
            return pltpu.async_copy(table.at[idxv[p]], buf[p], gsem[p])

        def gwait(p):
            pltpu.make_async_copy(table.at[idxv[p]], buf[p], gsem[p]).wait()

        def transpose(p):
            for c in range(32):
                vals = [plsc.load_gather(buf[p], [rows[lg], cols[c]])
                        for lg in range(8)]
                for lg in range(8):
                    tr[p][c // 8, c % 8, pl.ds(lg * 16, 16)] = vals[lg]

        def wstart(u, p):
            s, bt = u // 8, u % 8
            pltpu.async_copy(tr[p], out5.at[2 * s, :, bt, :, :], wsem[p])
            pltpu.async_copy(tr[p], out5.at[2 * s + 1, :, bt, :, :], wsem[p])

        def wwait(u, p):
            s, bt = u // 8, u % 8
            for _ in range(2):
                pltpu.make_async_copy(
                    tr[p], out5.at[2 * s, :, bt, :, :], wsem[p]).wait()

        ic0 = icopy_start(base + 0, 0)
        ic1 = icopy_start(base + 1, 1)
        ic0.wait()
        gstart(0)
        ic1.wait()
        gstart(1)

        for pp in range(2):
            u = base + pp
            gwait(pp)
            icopy_start(base + pp + 2, pp).wait()
            transpose(pp)
            wstart(u, pp)
            gstart(pp)

        def body(k, carry):
            for pp in range(2):
                i = 2 + k * 2 + pp
                u = base + i
                nxt = base + jnp.minimum(i + 2, _PER_W - 1)
                gwait(pp)
                ic = icopy_start(nxt, pp)
                wwait(u, pp)
                transpose(pp)
                wstart(u, pp)
                ic.wait()
                gstart(pp)
            return carry

        lax.fori_loop(0, (_PER_W - 2) // 2, body, 0)

        gwait(0)
        gwait(1)
        wwait(base + _PER_W - 2, 0)
        wwait(base + _PER_W - 1, 1)

    return gather_expand


_GATHER_EXPAND = _make_kernel()


def kernel(input, embedding_weight):
    idx4 = jnp.transpose(input.astype(jnp.int32).reshape(8, 128, 25, 8),
                         (2, 0, 3, 1))
    out5 = _GATHER_EXPAND(embedding_weight, idx4)
    return out5.transpose(2, 4, 0, 1, 3).reshape(1024, 400, 32)


# speedup vs baseline: 1.0981x; 1.0440x over previous
import functools

import jax
import jax.numpy as jnp
from jax import lax
from jax.experimental import pallas as pl
from jax.experimental.pallas import tpu as pltpu
from jax.experimental.pallas import tpu_sc as plsc

_NC = 2
_NS = 16
_NW = _NC * _NS

_NUNIT = 200 * 8
_PER_W = _NUNIT // _NW


def _make_kernel():
    mesh = plsc.VectorSubcoreMesh(
        core_axis_name="c", subcore_axis_name="s",
        num_cores=_NC, num_subcores=_NS,
    )

    @functools.partial(
        pl.kernel,
        mesh=mesh,
        out_type=jax.ShapeDtypeStruct((400, 4, 8, 8, 128), jnp.float32),
        scratch_types=[
            pltpu.VMEM((128,), jnp.int32),
            pltpu.VMEM((128,), jnp.int32),
            pltpu.VMEM((128, 32), jnp.float32),
            pltpu.VMEM((128, 32), jnp.float32),
            pltpu.VMEM((4, 8, 128), jnp.float32),
            pltpu.VMEM((4, 8, 128), jnp.float32),
            pltpu.SemaphoreType.DMA,
            pltpu.SemaphoreType.DMA,
            pltpu.SemaphoreType.DMA,
            pltpu.SemaphoreType.DMA,
            pltpu.SemaphoreType.DMA,
            pltpu.SemaphoreType.DMA,
        ],
        compiler_params=pltpu.CompilerParams(use_tc_tiling_on_sc=False,
                                             needs_layout_passes=False),
    )
    def gather_expand(table, idx4, out5, idxv0, idxv1, buf0, buf1, tr0, tr1,
                      isem0, isem1, gsem0, gsem1, wsem0, wsem1):
        wid = lax.axis_index("s") * _NC + lax.axis_index("c")
        base = wid * _PER_W

        idxv = (idxv0, idxv1)
        buf = (buf0, buf1)
        tr = (tr0, tr1)
        isem = (isem0, isem1)
        gsem = (gsem0, gsem1)
        wsem = (wsem0, wsem1)

        lane = lax.iota(jnp.int32, 16)
        dgv = [(lane + h * 16) // 8 for h in range(2)]
        d8v = [(lane + h * 16) % 8 for h in range(2)]

        def icopy_start(u, p):
            s, bt = u // 8, u % 8
            return pltpu.async_copy(idx4.at[s // 8, bt, s % 8, :], idxv[p],
                                    isem[p])

        def gstart(p):
            return pltpu.async_copy(table.at[idxv[p]], buf[p], gsem[p])

        def gwait(p):
            pltpu.make_async_copy(table.at[idxv[p]], buf[p], gsem[p]).wait()

        def transpose(p):
            for r in range(128):
                rv = jnp.full((16,), r, jnp.int32)
                for h in range(2):
                    vals = buf[p][r, pl.ds(h * 16, 16)]
                    plsc.store_scatter(tr[p], [dgv[h], d8v[h], rv], vals)

        def wstart(u, p):
            s, bt = u // 8, u % 8
            pltpu.async_copy(tr[p], out5.at[2 * s, :, bt, :, :], wsem[p])
            pltpu.async_copy(tr[p], out5.at[2 * s + 1, :, bt, :, :], wsem[p])

        def wwait(u, p):
            s, bt = u // 8, u % 8
            for _ in range(2):
                pltpu.make_async_copy(
                    tr[p], out5.at[2 * s, :, bt, :, :], wsem[p]).wait()

        ic0 = icopy_start(base + 0, 0)
        ic1 = icopy_start(base + 1, 1)
        ic0.wait()
        gstart(0)
        ic1.wait()
        gstart(1)

        for pp in range(2):
            u = base + pp
            gwait(pp)
            icopy_start(base + pp + 2, pp).wait()
            transpose(pp)
            wstart(u, pp)
            gstart(pp)

        def body(k, carry):
            for pp in range(2):
                i = 2 + k * 2 + pp
                u = base + i
                nxt = base + jnp.minimum(i + 2, _PER_W - 1)
                gwait(pp)
                ic = icopy_start(nxt, pp)
                wwait(u, pp)
                transpose(pp)
                wstart(u, pp)
                ic.wait()
                gstart(pp)
            return carry

        lax.fori_loop(0, (_PER_W - 2) // 2, body, 0)

        gwait(0)
        gwait(1)
        wwait(base + _PER_W - 2, 0)
        wwait(base + _PER_W - 1, 1)

    return gather_expand


_GATHER_EXPAND = _make_kernel()


def kernel(input, embedding_weight):
    idx4 = jnp.transpose(input.astype(jnp.int32).reshape(8, 128, 25, 8),
                         (2, 0, 3, 1))
    out5 = _GATHER_EXPAND(embedding_weight, idx4)
    return out5.transpose(2, 4, 0, 1, 3).reshape(1024, 400, 32)


# speedup vs baseline: 1.1240x; 1.0236x over previous
import functools

import jax
import jax.numpy as jnp
from jax import lax
from jax.experimental import pallas as pl
from jax.experimental.pallas import tpu as pltpu
from jax.experimental.pallas import tpu_sc as plsc

_NC = 2
_NS = 16
_NW = _NC * _NS

_NUNIT = 200 * 8
_PER_W = _NUNIT // _NW


def _make_kernel():
    mesh = plsc.VectorSubcoreMesh(
        core_axis_name="c", subcore_axis_name="s",
        num_cores=_NC, num_subcores=_NS,
    )

    @functools.partial(
        pl.kernel,
        mesh=mesh,
        out_type=jax.ShapeDtypeStruct((400, 4, 8, 8, 128), jnp.float32),
        scratch_types=[
            pltpu.VMEM((128,), jnp.int32),
            pltpu.VMEM((128,), jnp.int32),
            pltpu.VMEM((128, 32), jnp.float32),
            pltpu.VMEM((128, 32), jnp.float32),
            pltpu.VMEM((4, 8, 128), jnp.float32),
            pltpu.VMEM((4, 8, 128), jnp.float32),
            pltpu.SemaphoreType.DMA,
            pltpu.SemaphoreType.DMA,
            pltpu.SemaphoreType.DMA,
            pltpu.SemaphoreType.DMA,
            pltpu.SemaphoreType.DMA,
            pltpu.SemaphoreType.DMA,
        ],
        compiler_params=pltpu.CompilerParams(use_tc_tiling_on_sc=False,
                                             needs_layout_passes=False),
    )
    def gather_expand(table, idx4, out5, idxv0, idxv1, buf0, buf1, tr0, tr1,
                      isem0, isem1, gsem0, gsem1, wsem0, wsem1):
        wid = lax.axis_index("s") * _NC + lax.axis_index("c")
        base = wid * _PER_W

        idxv = (idxv0, idxv1)
        buf = (buf0, buf1)
        tr = (tr0, tr1)
        isem = (isem0, isem1)
        gsem = (gsem0, gsem1)
        wsem = (wsem0, wsem1)

        lane = lax.iota(jnp.int32, 16)
        dgv = [(lane + h * 16) // 8 for h in range(2)]
        d8v = [(lane + h * 16) % 8 for h in range(2)]

        def icopy_start(u, p):
            s, bt = u // 8, u % 8
            return pltpu.async_copy(idx4.at[s // 8, bt, s % 8, :], idxv[p],
                                    isem[p])

        def gstart(p):
            return pltpu.async_copy(table.at[idxv[p]], buf[p], gsem[p])

        def gwait(p):
            pltpu.make_async_copy(table.at[idxv[p]], buf[p], gsem[p]).wait()

        def transpose(p):
            for r0 in range(0, 128, 8):
                batch = [(r, h, buf[p][r, pl.ds(h * 16, 16)])
                         for r in range(r0, r0 + 8) for h in range(2)]
                for r, h, vals in batch:
                    rv = jnp.full((16,), r, jnp.int32)
                    plsc.store_scatter(tr[p], [dgv[h], d8v[h], rv], vals)

        def wstart(u, p):
            s, bt = u // 8, u % 8
            pltpu.async_copy(tr[p], out5.at[2 * s, :, bt, :, :], wsem[p])
            pltpu.async_copy(tr[p], out5.at[2 * s + 1, :, bt, :, :], wsem[p])

        def wwait(u, p):
            s, bt = u // 8, u % 8
            for _ in range(2):
                pltpu.make_async_copy(
                    tr[p], out5.at[2 * s, :, bt, :, :], wsem[p]).wait()

        ic0 = icopy_start(base + 0, 0)
        ic1 = icopy_start(base + 1, 1)
        ic0.wait()
        gstart(0)
        ic1.wait()
        gstart(1)

        for pp in range(2):
            u = base + pp
            gwait(pp)
            icopy_start(base + pp + 2, pp).wait()
            transpose(pp)
            wstart(u, pp)
            gstart(pp)

        def body(k, carry):
            for pp in range(2):
                i = 2 + k * 2 + pp
                u = base + i
                nxt = base + jnp.minimum(i + 2, _PER_W - 1)
                gwait(pp)
                ic = icopy_start(nxt, pp)
                wwait(u, pp)
                transpose(pp)
                wstart(u, pp)
                ic.wait()
                gstart(pp)
            return carry

        lax.fori_loop(0, (_PER_W - 2) // 2, body, 0)

        gwait(0)
        gwait(1)
        wwait(base + _PER_W - 2, 0)
        wwait(base + _PER_W - 1, 1)

    return gather_expand


_GATHER_EXPAND = _make_kernel()


def kernel(input, embedding_weight):
    idx4 = jnp.transpose(input.astype(jnp.int32).reshape(8, 128, 25, 8),
                         (2, 0, 3, 1))
    out5 = _GATHER_EXPAND(embedding_weight, idx4)
    return out5.transpose(2, 4, 0, 1, 3).reshape(1024, 400, 32)
